# baseline (speedup 1.0000x reference)
import functools
import jax
import jax.numpy as jnp
from jax import lax
from jax.experimental import pallas as pl
from jax.experimental.pallas import tpu as pltpu
from jax.experimental.pallas import tpu_sc as plsc

_BATCH, _SEQ, _DIM = 4, 1024, 1024
_NW = 32
_ROWS = _BATCH * _SEQ
_RPW = _ROWS // _NW
_RBLK = 32
_NBLK = _RPW // _RBLK
_L = 16


def _sc_body(x_hbm, tab_hbm, out_hbm, bufx, buft, semx, semt, semo):
    c = lax.axis_index("c")
    s = lax.axis_index("s")
    wid = s * 2 + c
    base = wid * _RPW
    trow = (wid % (_SEQ // _RPW)) * _RPW

    def blk_body(blk, carry):
        off = (base + blk * _RBLK) * _DIM
        toff = (trow + blk * _RBLK) * _DIM
        cpx = pltpu.async_copy(x_hbm.at[pl.ds(off, _RBLK * _DIM)], bufx, semx)
        cpt = pltpu.async_copy(tab_hbm.at[pl.ds(toff, _RBLK * _DIM)], buft, semt)
        cpx.wait()
        cpt.wait()

        def add_body(i, c2):
            j = i * (8 * _L)
            for u in range(8):
                t = buft[pl.ds(j + u * _L, _L)]
                plsc.addupdate(bufx.at[pl.ds(j + u * _L, _L)], t)
            return c2

        lax.fori_loop(0, _RBLK * _DIM // (8 * _L), add_body, 0)
        pltpu.async_copy(bufx, out_hbm.at[pl.ds(off, _RBLK * _DIM)], semo).wait()
        return carry

    lax.fori_loop(0, _NBLK, blk_body, 0)


@functools.partial(jax.jit, static_argnums=())
def _sc_add(x_flat, tab_flat):
    mesh = plsc.VectorSubcoreMesh(core_axis_name="c", subcore_axis_name="s")
    f = pl.kernel(
        _sc_body,
        mesh=mesh,
        out_type=jax.ShapeDtypeStruct((_ROWS * _DIM,), jnp.float32),
        scratch_types=[
            pltpu.VMEM((_RBLK * _DIM,), jnp.float32),
            pltpu.VMEM((_RBLK * _DIM,), jnp.float32),
            pltpu.SemaphoreType.DMA,
            pltpu.SemaphoreType.DMA,
            pltpu.SemaphoreType.DMA,
        ],
    )
    return f(x_flat, tab_flat)


def kernel(x, pos_emb_table):
    batch, seq, dim = x.shape
    out = _sc_add(x.reshape(-1), pos_emb_table.reshape(-1))
    return out.reshape(batch, seq, dim)


# speedup vs baseline: 1.4491x; 1.4491x over previous
import functools
import jax
import jax.numpy as jnp
from jax import lax
from jax.experimental import pallas as pl
from jax.experimental.pallas import tpu as pltpu
from jax.experimental.pallas import tpu_sc as plsc

_BATCH, _SEQ, _DIM = 4, 1024, 1024
_NW = 32
_ROWS = _BATCH * _SEQ
_RPW = _ROWS // _NW
_RBLK = 32
_NBLK = _RPW // _RBLK
_L = 16


def _sc_body(x_hbm, tab_hbm, out_hbm, bufx, buft, semx, semt, semo):
    c = lax.axis_index("c")
    s = lax.axis_index("s")
    wid = s * 2 + c
    base = wid * _RPW
    trow = (wid % (_SEQ // _RPW)) * _RPW

    def blk_body(blk, carry):
        r0 = base + blk * _RBLK
        t0 = trow + blk * _RBLK
        cpx = pltpu.async_copy(x_hbm.at[pl.ds(r0, _RBLK), :], bufx, semx)
        cpt = pltpu.async_copy(tab_hbm.at[pl.ds(t0, _RBLK), :], buft, semt)
        cpx.wait()
        cpt.wait()

        def add_body(i, c2):
            for u in range(8):
                j = (i * 8 + u) * _L
                r = j // _DIM
                col = j % _DIM
                t = buft[r, pl.ds(col, _L)]
                plsc.addupdate(bufx.at[r, pl.ds(col, _L)], t)
            return c2

        lax.fori_loop(0, _RBLK * _DIM // (8 * _L), add_body, 0)
        pltpu.async_copy(bufx, out_hbm.at[pl.ds(r0, _RBLK), :], semo).wait()
        return carry

    lax.fori_loop(0, _NBLK, blk_body, 0)


@jax.jit
def _sc_add(x2d, tab2d):
    mesh = plsc.VectorSubcoreMesh(core_axis_name="c", subcore_axis_name="s")
    f = pl.kernel(
        _sc_body,
        mesh=mesh,
        out_type=jax.ShapeDtypeStruct((_ROWS, _DIM), jnp.float32),
        scratch_types=[
            pltpu.VMEM((_RBLK, _DIM), jnp.float32),
            pltpu.VMEM((_RBLK, _DIM), jnp.float32),
            pltpu.SemaphoreType.DMA,
            pltpu.SemaphoreType.DMA,
            pltpu.SemaphoreType.DMA,
        ],
        compiler_params=pltpu.CompilerParams(use_tc_tiling_on_sc=True),
    )
    return f(x2d, tab2d)


def kernel(x, pos_emb_table):
    batch, seq, dim = x.shape
    out = _sc_add(x.reshape(batch * seq, dim), pos_emb_table)
    return out.reshape(batch, seq, dim)


# speedup vs baseline: 1.7225x; 1.1887x over previous
import jax
import jax.numpy as jnp
from jax import lax
from jax.experimental import pallas as pl
from jax.experimental.pallas import tpu as pltpu
from jax.experimental.pallas import tpu_sc as plsc

_BATCH, _SEQ, _DIM = 4, 1024, 1024
_NW = 32
_RPW = _SEQ // _NW
_L = 16
_VECS = _RPW * _DIM // _L
_UNROLL = 8


def _sc_body(x_hbm, tab_hbm, out_hbm, buft, bufx0, bufx1, semt,
             semi0, semi1, semo0, semo1):
    c = lax.axis_index("c")
    s = lax.axis_index("s")
    wid = s * 2 + c
    t0 = wid * _RPW
    bufx = (bufx0, bufx1)
    semi = (semi0, semi1)
    semo = (semo0, semo1)

    tab_cp = pltpu.async_copy(tab_hbm.at[pl.ds(t0, _RPW), :], buft, semt)
    in_cp = [None, None]
    out_cp = [None, None]
    in_cp[0] = pltpu.async_copy(x_hbm.at[pl.ds(t0, _RPW), :], bufx0, semi0)
    tab_cp.wait()

    for b in range(_BATCH):
        cur = b % 2
        nxt = (b + 1) % 2
        in_cp[cur].wait()
        if b + 1 < _BATCH:
            if out_cp[nxt] is not None:
                out_cp[nxt].wait()
            r_next = (b + 1) * _SEQ + t0
            in_cp[nxt] = pltpu.async_copy(
                x_hbm.at[pl.ds(r_next, _RPW), :], bufx[nxt], semi[nxt])

        buf = bufx[cur]

        def add_body(i, carry, buf=buf):
            for u in range(_UNROLL):
                j = (i * _UNROLL + u) * _L
                r = j // _DIM
                col = j % _DIM
                t = buft[r, pl.ds(col, _L)]
                plsc.addupdate(buf.at[r, pl.ds(col, _L)], t)
            return carry

        lax.fori_loop(0, _VECS // _UNROLL, add_body, 0)
        r_cur = b * _SEQ + t0
        out_cp[cur] = pltpu.async_copy(
            buf, out_hbm.at[pl.ds(r_cur, _RPW), :], semo[cur])

    out_cp[0].wait()
    out_cp[1].wait()


@jax.jit
def _sc_add(x2d, tab2d):
    mesh = plsc.VectorSubcoreMesh(core_axis_name="c", subcore_axis_name="s")
    f = pl.kernel(
        _sc_body,
        mesh=mesh,
        out_type=jax.ShapeDtypeStruct((_BATCH * _SEQ, _DIM), jnp.float32),
        scratch_types=[
            pltpu.VMEM((_RPW, _DIM), jnp.float32),
            pltpu.VMEM((_RPW, _DIM), jnp.float32),
            pltpu.VMEM((_RPW, _DIM), jnp.float32),
            pltpu.SemaphoreType.DMA,
            pltpu.SemaphoreType.DMA,
            pltpu.SemaphoreType.DMA,
            pltpu.SemaphoreType.DMA,
            pltpu.SemaphoreType.DMA,
        ],
        compiler_params=pltpu.CompilerParams(use_tc_tiling_on_sc=True),
    )
    return f(x2d, tab2d)


def kernel(x, pos_emb_table):
    batch, seq, dim = x.shape
    out = _sc_add(x.reshape(batch * seq, dim), pos_emb_table)
    return out.reshape(batch, seq, dim)
